# baseline (speedup 1.0000x reference)
import functools

import jax
import jax.numpy as jnp
from jax import lax
from jax.experimental import pallas as pl
from jax.experimental.pallas import tpu as pltpu
from jax.experimental.pallas import tpu_sc as plsc

NC = 2
NS = 16
NW = NC * NS
LANES = 16

CHUNK = 512
GSUB = 128


def _sc_unified_gather(flat_idx, table, n_feat, batch):
    total = n_feat * batch
    q, d = table.shape
    per_w = total // NW
    n_chunks = per_w // CHUNK
    assert per_w % CHUNK == 0 and CHUNK % GSUB == 0 and CHUNK % LANES == 0
    batch_shift = batch.bit_length() - 1
    assert batch == (1 << batch_shift)

    mesh = plsc.VectorSubcoreMesh(core_axis_name="c", subcore_axis_name="s")

    @functools.partial(
        pl.kernel,
        mesh=mesh,
        out_type=jax.ShapeDtypeStruct((total, d), jnp.float32),
        compiler_params=pltpu.CompilerParams(use_tc_tiling_on_sc=False),
        scratch_types=[
            pltpu.VMEM((CHUNK,), jnp.int32),
            pltpu.VMEM((CHUNK, d), jnp.float32),
            pltpu.SemaphoreType.DMA,
        ],
    )
    def sc_kernel(idx_hbm, table_hbm, out_hbm, idx_v, rows_v, sem):
        wid = lax.axis_index("s") * NC + lax.axis_index("c")
        base = wid * per_w

        @pl.loop(0, n_chunks)
        def _(j):
            pos0 = base + j * CHUNK
            pltpu.sync_copy(idx_hbm.at[pl.ds(pos0, CHUNK)], idx_v)
            for t in range(CHUNK // LANES):
                p = pos0 + t * LANES + lax.broadcasted_iota(jnp.int32, (LANES,), 0)
                salt = (p >> batch_shift) * 7919
                raw = idx_v[pl.ds(t * LANES, LANES)]
                idx_v[pl.ds(t * LANES, LANES)] = (raw * 31 + salt) % q
            copies = []
            for g in range(CHUNK // GSUB):
                copies.append(pltpu.async_copy(
                    table_hbm.at[idx_v.at[pl.ds(g * GSUB, GSUB)]],
                    rows_v.at[pl.ds(g * GSUB, GSUB)],
                    sem,
                ))
            for c in copies:
                c.wait()
            pltpu.sync_copy(rows_v, out_hbm.at[pl.ds(pos0, CHUNK)])

    return sc_kernel(flat_idx, table)


def kernel(inputs, table):
    n_feat, batch, _ = inputs.shape
    d = table.shape[1]
    flat_idx = inputs.reshape(n_feat * batch)
    out = _sc_unified_gather(flat_idx, table, n_feat, batch)
    out = out.reshape(n_feat, batch, d)
    return tuple(out[i] for i in range(n_feat))


# speedup vs baseline: 1.1037x; 1.1037x over previous
import functools

import jax
import jax.numpy as jnp
from jax import lax
from jax.experimental import pallas as pl
from jax.experimental.pallas import tpu as pltpu
from jax.experimental.pallas import tpu_sc as plsc

NC = 2
NS = 16
NW = NC * NS
LANES = 16

CHUNK = 512
GSUB = 128
NBUF = 2


def _sc_unified_gather(flat_idx, table, n_feat, batch):
    total = n_feat * batch
    q, d = table.shape
    per_w = total // NW
    n_chunks = per_w // CHUNK
    assert per_w % CHUNK == 0 and CHUNK % GSUB == 0 and CHUNK % LANES == 0
    assert n_chunks % NBUF == 0 and n_chunks >= 2 * NBUF
    batch_shift = batch.bit_length() - 1
    assert batch == (1 << batch_shift)

    mesh = plsc.VectorSubcoreMesh(core_axis_name="c", subcore_axis_name="s")

    @functools.partial(
        pl.kernel,
        mesh=mesh,
        out_type=jax.ShapeDtypeStruct((total, d), jnp.float32),
        compiler_params=pltpu.CompilerParams(use_tc_tiling_on_sc=False),
        scratch_types=[
            pltpu.VMEM((per_w,), jnp.int32),
            pltpu.VMEM((CHUNK, d), jnp.float32),
            pltpu.VMEM((CHUNK, d), jnp.float32),
            pltpu.SemaphoreType.DMA,
            pltpu.SemaphoreType.DMA,
        ],
    )
    def sc_kernel(idx_hbm, table_hbm, out_hbm, idx_v, rows0, rows1, sem0, sem1):
        rows = (rows0, rows1)
        sems = (sem0, sem1)
        wid = lax.axis_index("s") * NC + lax.axis_index("c")
        base = wid * per_w

        pltpu.sync_copy(idx_hbm.at[pl.ds(base, per_w)], idx_v)

        def hash_chunk(c):
            off = c * CHUNK

            @pl.loop(0, CHUNK, step=LANES)
            def _(t):
                sl = pl.ds(off + t, LANES)
                p = base + off + t + lax.broadcasted_iota(
                    jnp.int32, (LANES,), 0)
                salt = (p >> batch_shift) * 7919
                idx_v[sl] = (idx_v[sl] * 31 + salt) % q

        def issue_gathers(c, b):
            off = c * CHUNK
            for g in range(CHUNK // GSUB):
                pltpu.async_copy(
                    table_hbm.at[idx_v.at[pl.ds(off + g * GSUB, GSUB)]],
                    rows[b].at[pl.ds(g * GSUB, GSUB)],
                    sems[b],
                )

        def wait_gathers(b):
            pltpu.make_async_copy(
                table_hbm.at[pl.ds(0, CHUNK)], rows[b], sems[b]).wait()

        def write_out(c, b):
            pltpu.sync_copy(rows[b], out_hbm.at[pl.ds(base + c * CHUNK, CHUNK)])

        for b in range(NBUF):
            hash_chunk(b)
            issue_gathers(b, b)

        @pl.loop(NBUF, n_chunks, step=NBUF)
        def _(c):
            for b in range(NBUF):
                wait_gathers(b)
                write_out(c + b - NBUF, b)
                hash_chunk(c + b)
                issue_gathers(c + b, b)

        for b in range(NBUF):
            wait_gathers(b)
            write_out(n_chunks - NBUF + b, b)

    return sc_kernel(flat_idx, table)


def kernel(inputs, table):
    n_feat, batch, _ = inputs.shape
    d = table.shape[1]
    flat_idx = inputs.reshape(n_feat * batch)
    out = _sc_unified_gather(flat_idx, table, n_feat, batch)
    out = out.reshape(n_feat, batch, d)
    return tuple(out[i] for i in range(n_feat))


# speedup vs baseline: 1.4638x; 1.3263x over previous
import functools

import jax
import jax.numpy as jnp
from jax import lax
from jax.experimental import pallas as pl
from jax.experimental.pallas import tpu as pltpu
from jax.experimental.pallas import tpu_sc as plsc

NC = 2
NS = 16
NW = NC * NS
LANES = 16
GSUB = 128


def _sc_unified_gather(idx2d, table, n_feat, batch):
    q, d = table.shape
    chunk = batch // NW
    assert batch % NW == 0 and chunk % GSUB == 0 and chunk % LANES == 0

    mesh = plsc.VectorSubcoreMesh(core_axis_name="c", subcore_axis_name="s")

    @functools.partial(
        pl.kernel,
        mesh=mesh,
        out_type=[jax.ShapeDtypeStruct((batch, d), jnp.float32)
                  for _ in range(n_feat)],
        compiler_params=pltpu.CompilerParams(use_tc_tiling_on_sc=False),
        scratch_types=[
            pltpu.VMEM((chunk,), jnp.int32),
            pltpu.VMEM((chunk,), jnp.int32),
            pltpu.VMEM((chunk, d), jnp.float32),
            pltpu.VMEM((chunk, d), jnp.float32),
        ] + [pltpu.SemaphoreType.DMA] * 6,
    )
    def sc_kernel(idx_hbm, table_hbm, *rest):
        outs = rest[:n_feat]
        idxb = rest[n_feat:n_feat + 2]
        rows = rest[n_feat + 2:n_feat + 4]
        semi = rest[n_feat + 4:n_feat + 6]
        semg = rest[n_feat + 6:n_feat + 8]
        semw = rest[n_feat + 8:n_feat + 10]
        wid = lax.axis_index("s") * NC + lax.axis_index("c")
        base = wid * chunk

        def issue_idx_load(f, b):
            return pltpu.async_copy(
                idx_hbm.at[f, pl.ds(base, chunk)], idxb[b], semi[b])

        def hash_chunk(f, b):
            salt = f * 7919

            @pl.loop(0, chunk, step=LANES)
            def _(t):
                sl = pl.ds(t, LANES)
                idxb[b][sl] = (idxb[b][sl] * 31 + salt) % q

        def issue_gathers(f, b):
            return [
                pltpu.async_copy(
                    table_hbm.at[idxb[b].at[pl.ds(g * GSUB, GSUB)]],
                    rows[b].at[pl.ds(g * GSUB, GSUB)],
                    semg[b],
                )
                for g in range(chunk // GSUB)
            ]

        def issue_write(f, b):
            return pltpu.async_copy(
                rows[b], outs[f].at[pl.ds(base, chunk)], semw[b])

        ih = [None] * (n_feat + 2)
        gh = [None] * (n_feat + 1)
        wh = [None] * (n_feat + 1)
        ih[0] = issue_idx_load(0, 0)
        ih[1] = issue_idx_load(1, 1)
        ih[0].wait()
        hash_chunk(0, 0)
        gh[0] = issue_gathers(0, 0)

        for f in range(n_feat):
            b = f % 2
            for h in gh[f]:
                h.wait()
            wh[f] = issue_write(f, b)
            if f + 2 < n_feat:
                ih[f + 2] = issue_idx_load(f + 2, b)
            if f + 1 < n_feat:
                if wh[f - 1] is not None:
                    wh[f - 1].wait()
                ih[f + 1].wait()
                hash_chunk(f + 1, 1 - b)
                gh[f + 1] = issue_gathers(f + 1, 1 - b)
        wh[n_feat - 1].wait()
        wh[n_feat - 2].wait()

    return sc_kernel(idx2d, table)


def kernel(inputs, table):
    n_feat, batch, _ = inputs.shape
    idx2d = inputs.reshape(n_feat, batch)
    return tuple(_sc_unified_gather(idx2d, table, n_feat, batch))
